# Initial kernel scaffold; baseline (speedup 1.0000x reference)
import jax
import jax.numpy as jnp
from jax.experimental import pallas as pl


def kernel(x, edge_index, W1a, b1a, W1b, b1b, W2a, b2a, W2b, b2b, Wl, bl):
    raise NotImplementedError("write your pallas kernel here")


# speedup vs baseline: 3.2983x; 3.2983x over previous
import functools

import jax
import jax.numpy as jnp
from jax import lax
from jax.experimental import pallas as pl
from jax.experimental.pallas import tpu as pltpu
from jax.experimental.pallas import tpu_sc as plsc

N = 10000
D = 128
C = 64
E = 320000

NC = 2
NS = 16
NW = NC * NS

CHUNK = 128
CHUNKS = 80
E_PAD = NW * CHUNKS * CHUNK
N_PAD = 10112
ZR = N_PAD // NS


def _sc_agg_body(table, srcs, dsts, out, src_v, dst_v, rows_v, acc, sem):
  c = lax.axis_index("c")
  s = lax.axis_index("s")
  wid = s * NC + c

  def zrow(r, carry):
    for q in range(D // 16):
      rows_v[r, pl.ds(q * 16, 16)] = jnp.zeros((16,), jnp.float32)
    return carry

  lax.fori_loop(0, CHUNK, zrow, 0)

  zbase = s * ZR
  for k in range(ZR // CHUNK):
    pltpu.sync_copy(rows_v, acc.at[pl.ds(zbase + k * CHUNK, CHUNK)])
  rem = ZR % CHUNK
  if rem:
    pltpu.sync_copy(
        rows_v.at[pl.ds(0, rem)],
        acc.at[pl.ds(zbase + (ZR // CHUNK) * CHUNK, rem)],
    )
  plsc.subcore_barrier()

  pltpu.sync_copy(srcs.at[wid], src_v)
  pltpu.sync_copy(dsts.at[wid], dst_v)

  def chunk_body(j, carry):
    pltpu.async_copy(table.at[src_v.at[j]], rows_v, sem).wait()
    pltpu.sync_copy(rows_v, acc.at[dst_v.at[j]], add=True)
    return carry

  lax.fori_loop(0, CHUNKS, chunk_body, 0)
  plsc.subcore_barrier()

  pltpu.sync_copy(acc.at[pl.ds(zbase, ZR)], out.at[c, pl.ds(zbase, ZR)])


_sc_agg = pl.kernel(
    _sc_agg_body,
    out_type=jax.ShapeDtypeStruct((NC, N_PAD, D), jnp.float32),
    mesh=plsc.VectorSubcoreMesh(core_axis_name="c", subcore_axis_name="s"),
    scratch_types=[
        pltpu.VMEM((CHUNKS, CHUNK), jnp.int32),
        pltpu.VMEM((CHUNKS, CHUNK), jnp.int32),
        pltpu.VMEM((CHUNK, D), jnp.float32),
        pltpu.VMEM_SHARED((N_PAD, D), jnp.float32),
        pltpu.SemaphoreType.DMA,
    ],
)


def _mlp_body(final, x_ref, p0_ref, p1_ref, wa_ref, ba_ref, wb_ref, bb_ref,
              wl_ref, bl_ref, o_ref):
  h = x_ref[...] + p0_ref[...] + p1_ref[...]
  z = jnp.dot(h, wa_ref[...], preferred_element_type=jnp.float32) + ba_ref[...]
  z = jnp.maximum(z, 0.0)
  o = jnp.dot(z, wb_ref[...], preferred_element_type=jnp.float32) + bb_ref[...]
  o = jnp.maximum(o, 0.0)
  if final:
    o = jnp.dot(o, wl_ref[...], preferred_element_type=jnp.float32) + bl_ref[...]
  o_ref[...] = o


_BLK = 1000


def _tc_mlp(x, p0, p1, wa, ba, wb, bb, wl, bl, final):
  out_c = C if final else D
  grid = (N // _BLK,)
  row_spec = pl.BlockSpec((_BLK, D), lambda i: (i, 0))
  full = lambda r, c: pl.BlockSpec((r, c), lambda i: (0, 0))
  return pl.pallas_call(
      functools.partial(_mlp_body, final),
      grid=grid,
      in_specs=[
          row_spec, row_spec, row_spec,
          full(D, D), full(1, D), full(D, D), full(1, D),
          full(D, C), full(1, C),
      ],
      out_specs=pl.BlockSpec((_BLK, out_c), lambda i: (i, 0)),
      out_shape=jax.ShapeDtypeStruct((N, out_c), jnp.float32),
  )(x, p0, p1, wa, ba.reshape(1, D), wb, bb.reshape(1, D),
    wl, bl.reshape(1, -1))


def kernel(x, edge_index, W1a, b1a, W1b, b1b, W2a, b2a, W2b, b2b, Wl, bl):
  src = edge_index[0]
  dst = edge_index[1]
  pad = E_PAD - E
  srcs = jnp.concatenate([src, jnp.zeros((pad,), jnp.int32)])
  dsts = jnp.concatenate([dst, jnp.full((pad,), N, jnp.int32)])
  srcs = srcs.reshape(NW, CHUNKS, CHUNK)
  dsts = dsts.reshape(NW, CHUNKS, CHUNK)

  agg1 = _sc_agg(x, srcs, dsts)
  h1 = _tc_mlp(x, agg1[0], agg1[1], W1a, b1a, W1b, b1b, Wl, bl, final=False)
  agg2 = _sc_agg(h1, srcs, dsts)
  return _tc_mlp(h1, agg2[0], agg2[1], W2a, b2a, W2b, b2b, Wl, bl, final=True)


# speedup vs baseline: 8.0899x; 2.4528x over previous
import functools

import jax
import jax.numpy as jnp
from jax import lax
from jax.experimental import pallas as pl
from jax.experimental.pallas import tpu as pltpu
from jax.experimental.pallas import tpu_sc as plsc

N = 10000
D = 128
C = 64
E = 320000

NC = 2
NS = 16
NW = NC * NS

CHUNK = 128
CHUNKS = 80
E_PAD = NW * CHUNKS * CHUNK
N_PAD = 10112
ZR = N_PAD // NS


def _sc_agg_body(table, srcs, dsts, out, src_v, dst_v, rows_v, acc, sem):
  c = lax.axis_index("c")
  s = lax.axis_index("s")
  wid = s * NC + c

  def zrow(r, carry):
    for q in range(D // 16):
      rows_v[r, pl.ds(q * 16, 16)] = jnp.zeros((16,), jnp.float32)
    return carry

  lax.fori_loop(0, CHUNK, zrow, 0)

  zbase = s * ZR
  for k in range(ZR // CHUNK):
    pltpu.sync_copy(rows_v, acc.at[pl.ds(zbase + k * CHUNK, CHUNK)])
  rem = ZR % CHUNK
  if rem:
    pltpu.sync_copy(
        rows_v.at[pl.ds(0, rem)],
        acc.at[pl.ds(zbase + (ZR // CHUNK) * CHUNK, rem)],
    )
  plsc.subcore_barrier()

  pltpu.sync_copy(srcs.at[wid], src_v)
  pltpu.sync_copy(dsts.at[wid], dst_v)

  def chunk_body(j, carry):
    pltpu.async_copy(table.at[src_v.at[j]], rows_v, sem).wait()
    pltpu.sync_copy(rows_v, acc.at[dst_v.at[j]], add=True)
    return carry

  lax.fori_loop(0, CHUNKS, chunk_body, 0)
  plsc.subcore_barrier()

  pltpu.sync_copy(acc.at[pl.ds(zbase, ZR)], out.at[c, pl.ds(zbase, ZR)])


_sc_agg = pl.kernel(
    _sc_agg_body,
    out_type=jax.ShapeDtypeStruct((NC, N_PAD, D), jnp.float32),
    mesh=plsc.VectorSubcoreMesh(core_axis_name="c", subcore_axis_name="s"),
    scratch_types=[
        pltpu.VMEM((CHUNKS, CHUNK), jnp.int32),
        pltpu.VMEM((CHUNKS, CHUNK), jnp.int32),
        pltpu.VMEM((CHUNK, D), jnp.float32),
        pltpu.VMEM_SHARED((N_PAD, D), jnp.float32),
        pltpu.SemaphoreType.DMA,
    ],
)


def _mlp_body(final, x_ref, p0_ref, p1_ref, wa_ref, ba_ref, wb_ref, bb_ref,
              wl_ref, bl_ref, o_ref):
  h = x_ref[...] + p0_ref[...] + p1_ref[...]
  z = jnp.dot(h, wa_ref[...], preferred_element_type=jnp.float32) + ba_ref[...]
  z = jnp.maximum(z, 0.0)
  o = jnp.dot(z, wb_ref[...], preferred_element_type=jnp.float32) + bb_ref[...]
  o = jnp.maximum(o, 0.0)
  if final:
    o = jnp.dot(o, wl_ref[...], preferred_element_type=jnp.float32) + bl_ref[...]
  o_ref[...] = o


_BLK = 1000


def _tc_mlp(x, p0, p1, wa, ba, wb, bb, wl, bl, final):
  out_c = C if final else D
  grid = (N // _BLK,)
  row_spec = pl.BlockSpec((_BLK, D), lambda i: (i, 0))
  full = lambda r, c: pl.BlockSpec((r, c), lambda i: (0, 0))
  return pl.pallas_call(
      functools.partial(_mlp_body, final),
      grid=grid,
      in_specs=[
          row_spec, row_spec, row_spec,
          full(D, D), full(1, D), full(D, D), full(1, D),
          full(D, C), full(1, C),
      ],
      out_specs=pl.BlockSpec((_BLK, out_c), lambda i: (i, 0)),
      out_shape=jax.ShapeDtypeStruct((N, out_c), jnp.float32),
  )(x, p0, p1, wa, ba.reshape(1, D), wb, bb.reshape(1, D),
    wl, bl.reshape(1, -1))


def kernel(x, edge_index, W1a, b1a, W1b, b1b, W2a, b2a, W2b, b2b, Wl, bl):
  src = edge_index[0]
  dst = edge_index[1]
  pad = E_PAD - E
  r = jnp.arange(pad, dtype=jnp.int32)
  srcs = jnp.concatenate([src, r % N])
  dsts = jnp.concatenate([dst, N + r % (N_PAD - N)])
  srcs = srcs.reshape(NW, CHUNKS, CHUNK)
  dsts = dsts.reshape(NW, CHUNKS, CHUNK)

  agg1 = _sc_agg(x, srcs, dsts)
  h1 = _tc_mlp(x, agg1[0], agg1[1], W1a, b1a, W1b, b1b, Wl, bl, final=False)
  agg2 = _sc_agg(h1, srcs, dsts)
  return _tc_mlp(h1, agg2[0], agg2[1], W2a, b2a, W2b, b2b, Wl, bl, final=True)


# speedup vs baseline: 10.2044x; 1.2614x over previous
import functools

import jax
import jax.numpy as jnp
from jax import lax
from jax.experimental import pallas as pl
from jax.experimental.pallas import tpu as pltpu
from jax.experimental.pallas import tpu_sc as plsc

N = 10000
D = 128
C = 64
E = 320000

NC = 2
NS = 16
NW = NC * NS

EPT = E // NW
CHUNK = 128
CHUNKS = 80
PHASES = 2
PH = CHUNKS // PHASES
EPT_PAD = CHUNKS * CHUNK
N_PAD = 10016
ZR = 624


def _sc_agg_body(table, srcs, dsts, out, src_v, dst_v, rows, gsems, acc):
  c = lax.axis_index("c")
  s = lax.axis_index("s")
  wid = s * NC + c

  def zrow(r, carry):
    for q in range(D // 16):
      rows[0][r, pl.ds(q * 16, 16)] = jnp.zeros((16,), jnp.float32)
    return carry

  lax.fori_loop(0, CHUNK, zrow, 0)

  def zero_rows(base, length):
    done = 0
    while done < length:
      step = min(CHUNK, length - done)
      pltpu.sync_copy(rows[0].at[pl.ds(0, step)],
                      acc.at[pl.ds(base + done, step)])
      done += step

  zbase = s * ZR
  zero_rows(zbase, ZR)

  @pl.when(s == NS - 1)
  def _():
    zero_rows(NS * ZR, N_PAD - NS * ZR)

  plsc.subcore_barrier()

  def fire(j, b):
    pltpu.async_copy(table.at[src_v.at[j]], rows[b], gsems[b])

  def drain(b):
    pltpu.make_async_copy(table.at[src_v.at[0]], rows[b], gsems[b]).wait()

  for p in range(PHASES):
    pltpu.sync_copy(srcs.at[wid, pl.ds(p * PH, PH)], src_v)
    pltpu.sync_copy(dsts.at[wid, pl.ds(p * PH, PH)], dst_v)
    fire(0, 0)

    def pair_body(g, carry):
      for b in range(2):
        j = g * 2 + b
        drain(b)

        @pl.when(j + 1 < PH)
        def _():
          fire(j + 1, 1 - b)

        pltpu.sync_copy(rows[b], acc.at[dst_v.at[j]], add=True)
      return carry

    lax.fori_loop(0, PH // 2, pair_body, 0)

  plsc.subcore_barrier()

  pltpu.sync_copy(acc.at[pl.ds(zbase, ZR)], out.at[c, pl.ds(zbase, ZR)])

  @pl.when(s == NS - 1)
  def _():
    pltpu.sync_copy(acc.at[pl.ds(NS * ZR, N_PAD - NS * ZR)],
                    out.at[c, pl.ds(NS * ZR, N_PAD - NS * ZR)])


_sc_agg = pl.kernel(
    _sc_agg_body,
    out_type=jax.ShapeDtypeStruct((NC, N_PAD, D), jnp.float32),
    mesh=plsc.VectorSubcoreMesh(core_axis_name="c", subcore_axis_name="s"),
    scratch_types=[
        pltpu.VMEM((PH, CHUNK), jnp.int32),
        pltpu.VMEM((PH, CHUNK), jnp.int32),
        [pltpu.VMEM((CHUNK, D), jnp.float32) for _ in range(2)],
        [pltpu.SemaphoreType.DMA for _ in range(2)],
        pltpu.VMEM_SHARED((N_PAD, D), jnp.float32),
    ],
)


def _mlp_body(final, x_ref, p0_ref, p1_ref, wa_ref, ba_ref, wb_ref, bb_ref,
              wl_ref, bl_ref, o_ref):
  h = x_ref[...] + p0_ref[...] + p1_ref[...]
  z = jnp.dot(h, wa_ref[...], preferred_element_type=jnp.float32) + ba_ref[...]
  z = jnp.maximum(z, 0.0)
  o = jnp.dot(z, wb_ref[...], preferred_element_type=jnp.float32) + bb_ref[...]
  o = jnp.maximum(o, 0.0)
  if final:
    o = jnp.dot(o, wl_ref[...], preferred_element_type=jnp.float32) + bl_ref[...]
  o_ref[...] = o


_BLK = 1000


def _tc_mlp(x, p0, p1, wa, ba, wb, bb, wl, bl, final):
  out_c = C if final else D
  grid = (N // _BLK,)
  row_spec = pl.BlockSpec((_BLK, D), lambda i: (i, 0))
  full = lambda r, c: pl.BlockSpec((r, c), lambda i: (0, 0))
  return pl.pallas_call(
      functools.partial(_mlp_body, final),
      grid=grid,
      in_specs=[
          row_spec, row_spec, row_spec,
          full(D, D), full(1, D), full(D, D), full(1, D),
          full(D, C), full(1, C),
      ],
      out_specs=pl.BlockSpec((_BLK, out_c), lambda i: (i, 0)),
      out_shape=jax.ShapeDtypeStruct((N, out_c), jnp.float32),
  )(x, p0, p1, wa, ba.reshape(1, D), wb, bb.reshape(1, D),
    wl, bl.reshape(1, -1))


def kernel(x, edge_index, W1a, b1a, W1b, b1b, W2a, b2a, W2b, b2b, Wl, bl):
  src = edge_index[0]
  dst = edge_index[1]
  pad = EPT_PAD - EPT
  r = jnp.arange(NW * pad, dtype=jnp.int32).reshape(NW, pad)
  srcs = jnp.concatenate([src.reshape(NW, EPT), r % N], axis=1)
  dsts = jnp.concatenate([dst.reshape(NW, EPT), N + r % (N_PAD - N)], axis=1)
  srcs = srcs.reshape(NW, CHUNKS, CHUNK)
  dsts = dsts.reshape(NW, CHUNKS, CHUNK)

  agg1 = _sc_agg(x, srcs, dsts)
  h1 = _tc_mlp(x, agg1[0], agg1[1], W1a, b1a, W1b, b1b, Wl, bl, final=False)
  agg2 = _sc_agg(h1, srcs, dsts)
  return _tc_mlp(h1, agg2[0], agg2[1], W2a, b2a, W2b, b2b, Wl, bl, final=True)
